# baseline (speedup 1.0000x reference)
import functools

import jax
import jax.numpy as jnp
from jax import lax
from jax.experimental import pallas as pl
from jax.experimental.pallas import tpu as pltpu
from jax.experimental.pallas import tpu_sc as plsc

N = 10000
E = 320000
DIM = 128


_ROWS = 400
_GRID = N // _ROWS
_PROWS = 400
_PGRID = N // _PROWS


def _pre_body(x_ref, h_ref, c_ref, w3t_ref, uft_ref, ufb_ref, iou_ref, gc_ref):
    fh = jnp.dot(h_ref[...], uft_ref[...], preferred_element_type=jnp.float32)
    g = jax.nn.sigmoid(fh + ufb_ref[...])
    gc_ref[...] = g * c_ref[...]
    iou_ref[...] = jnp.dot(x_ref[...], w3t_ref[...],
                           preferred_element_type=jnp.float32)


def _pre(x, h, c, w3t, uft, ufb):
    return pl.pallas_call(
        _pre_body,
        grid=(_GRID,),
        in_specs=[
            pl.BlockSpec((_ROWS, DIM), lambda i: (i, 0)),
            pl.BlockSpec((_ROWS, DIM), lambda i: (i, 0)),
            pl.BlockSpec((_ROWS, DIM), lambda i: (i, 0)),
            pl.BlockSpec((DIM, 3 * DIM), lambda i: (0, 0)),
            pl.BlockSpec((DIM, DIM), lambda i: (0, 0)),
            pl.BlockSpec((1, DIM), lambda i: (0, 0)),
        ],
        out_specs=[
            pl.BlockSpec((_ROWS, 3 * DIM), lambda i: (i, 0)),
            pl.BlockSpec((_ROWS, DIM), lambda i: (i, 0)),
        ],
        out_shape=[
            jax.ShapeDtypeStruct((N, 3 * DIM), jnp.float32),
            jax.ShapeDtypeStruct((N, DIM), jnp.float32),
        ],
    )(x, h, c, w3t, uft, ufb)


_NSUB = 16
_CHUNK = 160
_NCHUNK = 125
_EDGES_PER_SUB = _NCHUNK * _CHUNK
_APAD = 10240
_ROWS_PER_SUB = _APAD // _NSUB

@functools.cache
def _build_sc_segment_sum():
    mesh = plsc.VectorSubcoreMesh(core_axis_name="c", subcore_axis_name="s")
    return pl.kernel(
        _sc_segment_sum_body,
        out_type=jax.ShapeDtypeStruct((2, _APAD, DIM), jnp.float32),
        mesh=mesh,
        scratch_types=[
            pltpu.VMEM((_CHUNK,), jnp.int32),
            pltpu.VMEM((_CHUNK,), jnp.int32),
            pltpu.VMEM((_CHUNK // 2,), jnp.int32),
            pltpu.VMEM((_CHUNK // 2,), jnp.int32),
            pltpu.VMEM((_CHUNK // 2,), jnp.int32),
            pltpu.VMEM((_CHUNK // 2,), jnp.int32),
            pltpu.VMEM((_CHUNK, DIM), jnp.float32),
            pltpu.VMEM((_CHUNK, DIM), jnp.float32),
            pltpu.VMEM_SHARED((_APAD, DIM), jnp.float32),
        ] + [pltpu.SemaphoreType.DMA] * 10,
    )


def _sc_segment_sum_body(h_hbm, gc_hbm, src_hbm, dst_hbm, out_hbm,
                         sbuf0, sbuf1, dbuf0l, dbuf0h, dbuf1l, dbuf1h,
                         rows0, rows1, acc,
                         gsem0, gsem1, ssem0, ssem1, is0, is1, id0, id1,
                         g2sem0, g2sem1):
    c = lax.axis_index("c")
    s = lax.axis_index("s")
    sbuf = (sbuf0, sbuf1)
    dbuf = ((dbuf0l, dbuf0h), (dbuf1l, dbuf1h))
    rows = (rows0, rows1)
    gsem = (gsem0, gsem1)
    ssem = (ssem0, ssem1)
    ssem2 = (g2sem0, g2sem1)
    isem = (is0, is1)
    idsem = (id0, id1)
    base = s * _EDGES_PER_SUB
    _H = _CHUNK // 2

    def src_start(t, b):
        pltpu.async_copy(src_hbm.at[pl.ds(base + t * _CHUNK, _CHUNK)],
                         sbuf[b], isem[b])

    def src_wait(b):
        pltpu.make_async_copy(src_hbm.at[pl.ds(base, _CHUNK)],
                              sbuf[b], isem[b]).wait()

    def dst_start(t, b):
        pltpu.async_copy(dst_hbm.at[pl.ds(base + t * _CHUNK, _H)],
                         dbuf[b][0], idsem[b])
        pltpu.async_copy(dst_hbm.at[pl.ds(base + t * _CHUNK + _H, _H)],
                         dbuf[b][1], idsem[b])

    def dst_wait(b):
        pltpu.make_async_copy(dst_hbm.at[pl.ds(base, _H)],
                              dbuf[b][0], idsem[b]).wait()
        pltpu.make_async_copy(dst_hbm.at[pl.ds(base, _H)],
                              dbuf[b][1], idsem[b]).wait()

    def _zero_body(i, _):
        r = i // (DIM // 16)
        k = (i % (DIM // 16)) * 16
        rows0[r, pl.ds(k, 16)] = jnp.zeros((16,), jnp.float32)
        return _

    lax.fori_loop(0, _CHUNK * (DIM // 16), _zero_body, None)

    def _zcopy_body(k, _):
        pltpu.sync_copy(rows0,
                        acc.at[pl.ds(s * _ROWS_PER_SUB + k * _CHUNK, _CHUNK)])
        return _

    lax.fori_loop(0, _ROWS_PER_SUB // _CHUNK, _zcopy_body, None)
    plsc.subcore_barrier()

    def _edge_loop(table_hbm):
        q = 32
        gs = (gsem, ssem2)

        def gather_start(b):
            for j in range(_CHUNK // q):
                pltpu.async_copy(table_hbm.at[sbuf[b].at[pl.ds(j * q, q)]],
                                 rows[b].at[pl.ds(j * q, q)], gs[j % 2][b])

        def gather_wait(b):
            for j in range(_CHUNK // q):
                pltpu.make_async_copy(table_hbm.at[sbuf[b].at[pl.ds(j * q, q)]],
                                      rows[b].at[pl.ds(j * q, q)],
                                      gs[j % 2][b]).wait()

        src_start(0, 0)
        dst_start(0, 0)
        src_start(1, 1)
        dst_start(1, 1)
        src_wait(0)
        dst_wait(0)
        gather_start(0)

        def _step(t, b, *, idx_next, gather_next):
            gather_wait(b)
            sc0 = pltpu.async_copy(rows[b].at[pl.ds(0, _H)],
                                   acc.at[dbuf[b][0]], ssem[b], add=True)
            sc1 = pltpu.async_copy(rows[b].at[pl.ds(_H, _H)],
                                   acc.at[dbuf[b][1]], ssem[b], add=True)
            if idx_next:
                src_start(t + 2, b)
            if gather_next:
                src_wait(1 - b)
                dst_wait(1 - b)
                gather_start(1 - b)
            sc0.wait()
            sc1.wait()
            if idx_next:
                dst_start(t + 2, b)

        def _edge_body(i, _):
            _step(i * 2, 0, idx_next=True, gather_next=True)
            _step(i * 2 + 1, 1, idx_next=True, gather_next=True)
            return _

        lax.fori_loop(0, (_NCHUNK - 3) // 2, _edge_body, None)
        _step(_NCHUNK - 3, 0, idx_next=True, gather_next=True)
        _step(_NCHUNK - 2, 1, idx_next=False, gather_next=True)
        _step(_NCHUNK - 1, 0, idx_next=False, gather_next=False)

    @pl.when(c == 0)
    def _():
        _edge_loop(h_hbm)

    @pl.when(c == 1)
    def _():
        _edge_loop(gc_hbm)

    plsc.subcore_barrier()

    def _wb_body(k, _):
        row = s * _ROWS_PER_SUB + k * _CHUNK
        pltpu.sync_copy(acc.at[pl.ds(row, _CHUNK)], rows0)
        pltpu.sync_copy(rows0, out_hbm.at[c, pl.ds(row, _CHUNK)])
        return _

    lax.fori_loop(0, _ROWS_PER_SUB // _CHUNK, _wb_body, None)


def _post_body(hs_ref, ca_ref, ix_ref, ut_ref, b_ref, h_ref, c_ref):
    sm = jnp.dot(hs_ref[0], ut_ref[...], preferred_element_type=jnp.float32)
    iou = ix_ref[...] + sm + b_ref[...]
    i = jax.nn.sigmoid(iou[:, :DIM])
    o = jax.nn.sigmoid(iou[:, DIM:2 * DIM])
    u = jnp.tanh(iou[:, 2 * DIM:])
    c_new = i * u + ca_ref[0]
    h_ref[...] = o * jnp.tanh(c_new)
    c_ref[...] = c_new


def _post(agg, iou_x, ut, b):
    return pl.pallas_call(
        _post_body,
        grid=(_PGRID,),
        in_specs=[
            pl.BlockSpec((1, _PROWS, DIM), lambda i: (0, i, 0)),
            pl.BlockSpec((1, _PROWS, DIM), lambda i: (1, i, 0)),
            pl.BlockSpec((_PROWS, 3 * DIM), lambda i: (i, 0)),
            pl.BlockSpec((DIM, 3 * DIM), lambda i: (0, 0)),
            pl.BlockSpec((1, 3 * DIM), lambda i: (0, 0)),
        ],
        out_specs=[
            pl.BlockSpec((_PROWS, DIM), lambda i: (i, 0)),
            pl.BlockSpec((_PROWS, DIM), lambda i: (i, 0)),
        ],
        out_shape=[
            jax.ShapeDtypeStruct((N, DIM), jnp.float32),
            jax.ShapeDtypeStruct((N, DIM), jnp.float32),
        ],
    )(agg, agg, iou_x, ut, b)


@jax.jit
def kernel(x, h, c, edge_index, W_iouf, U_iou, b_iou, U_f_W, U_f_b):
    src = edge_index[0].astype(jnp.int32)
    dst = edge_index[1].astype(jnp.int32)

    w3t = W_iouf[:3 * DIM].T
    uft = U_f_W.T
    ufb = U_f_b.reshape(1, DIM)
    ut = U_iou.T

    iou_x, gc = _pre(x, h, c, w3t, uft, ufb)
    agg = _build_sc_segment_sum()(h, gc, src, dst)
    h_new, c_new = _post(agg, iou_x, ut, b_iou)
    return h_new, c_new


# speedup vs baseline: 1.0030x; 1.0030x over previous
import functools

import jax
import jax.numpy as jnp
from jax import lax
from jax.experimental import pallas as pl
from jax.experimental.pallas import tpu as pltpu
from jax.experimental.pallas import tpu_sc as plsc

N = 10000
E = 320000
DIM = 128


_ROWS = 400
_GRID = N // _ROWS
_PROWS = 400
_PGRID = N // _PROWS


def _pre_body(x_ref, h_ref, c_ref, w3t_ref, uft_ref, ufb_ref, iou_ref, gc_ref):
    fh = jnp.dot(h_ref[...], uft_ref[...], preferred_element_type=jnp.float32)
    g = jax.nn.sigmoid(fh + ufb_ref[...])
    gc_ref[...] = g * c_ref[...]
    iou_ref[...] = jnp.dot(x_ref[...], w3t_ref[...],
                           preferred_element_type=jnp.float32)


def _pre(x, h, c, w3t, uft, ufb):
    return pl.pallas_call(
        _pre_body,
        grid=(_GRID,),
        in_specs=[
            pl.BlockSpec((_ROWS, DIM), lambda i: (i, 0)),
            pl.BlockSpec((_ROWS, DIM), lambda i: (i, 0)),
            pl.BlockSpec((_ROWS, DIM), lambda i: (i, 0)),
            pl.BlockSpec((DIM, 3 * DIM), lambda i: (0, 0)),
            pl.BlockSpec((DIM, DIM), lambda i: (0, 0)),
            pl.BlockSpec((1, DIM), lambda i: (0, 0)),
        ],
        out_specs=[
            pl.BlockSpec((_ROWS, 3 * DIM), lambda i: (i, 0)),
            pl.BlockSpec((_ROWS, DIM), lambda i: (i, 0)),
        ],
        out_shape=[
            jax.ShapeDtypeStruct((N, 3 * DIM), jnp.float32),
            jax.ShapeDtypeStruct((N, DIM), jnp.float32),
        ],
    )(x, h, c, w3t, uft, ufb)


_NSUB = 16
_CHUNK = 160
_NCHUNK = 125
_EDGES_PER_SUB = _NCHUNK * _CHUNK
_APAD = 10240
_ROWS_PER_SUB = _APAD // _NSUB

@functools.cache
def _build_sc_segment_sum():
    mesh = plsc.VectorSubcoreMesh(core_axis_name="c", subcore_axis_name="s")
    return pl.kernel(
        _sc_segment_sum_body,
        out_type=jax.ShapeDtypeStruct((2, _APAD, DIM), jnp.float32),
        mesh=mesh,
        scratch_types=[
            pltpu.VMEM((_CHUNK,), jnp.int32),
            pltpu.VMEM((_CHUNK,), jnp.int32),
            pltpu.VMEM((4, _CHUNK // 4), jnp.int32),
            pltpu.VMEM((4, _CHUNK // 4), jnp.int32),
            pltpu.VMEM((_CHUNK, DIM), jnp.float32),
            pltpu.VMEM((_CHUNK, DIM), jnp.float32),
            pltpu.VMEM_SHARED((_APAD, DIM), jnp.float32),
        ] + [pltpu.SemaphoreType.DMA] * 10,
    )


def _sc_segment_sum_body(h_hbm, gc_hbm, src_hbm, dst_hbm, out_hbm,
                         sbuf0, sbuf1, dbuf0, dbuf1,
                         rows0, rows1, acc,
                         gsem0, gsem1, ssem0, ssem1, is0, is1, id0, id1,
                         g2sem0, g2sem1):
    c = lax.axis_index("c")
    s = lax.axis_index("s")
    sbuf = (sbuf0, sbuf1)
    dbuf = (dbuf0, dbuf1)
    rows = (rows0, rows1)
    gsem = (gsem0, gsem1)
    ssem = (ssem0, ssem1)
    ssem2 = (g2sem0, g2sem1)
    isem = (is0, is1)
    idsem = (id0, id1)
    base = s * _EDGES_PER_SUB
    _Q = _CHUNK // 4

    def src_start(t, b):
        pltpu.async_copy(src_hbm.at[pl.ds(base + t * _CHUNK, _CHUNK)],
                         sbuf[b], isem[b])

    def src_wait(b):
        pltpu.make_async_copy(src_hbm.at[pl.ds(base, _CHUNK)],
                              sbuf[b], isem[b]).wait()

    def dst_start(t, b):
        for j in range(4):
            pltpu.async_copy(
                dst_hbm.at[pl.ds(base + t * _CHUNK + j * _Q, _Q)],
                dbuf[b].at[j], idsem[b])

    def dst_wait(b):
        for j in range(4):
            pltpu.make_async_copy(dst_hbm.at[pl.ds(base, _Q)],
                                  dbuf[b].at[j], idsem[b]).wait()

    def _zero_body(i, _):
        r = i // (DIM // 16)
        k = (i % (DIM // 16)) * 16
        rows0[r, pl.ds(k, 16)] = jnp.zeros((16,), jnp.float32)
        return _

    lax.fori_loop(0, _CHUNK * (DIM // 16), _zero_body, None)

    def _zcopy_body(k, _):
        pltpu.sync_copy(rows0,
                        acc.at[pl.ds(s * _ROWS_PER_SUB + k * _CHUNK, _CHUNK)])
        return _

    lax.fori_loop(0, _ROWS_PER_SUB // _CHUNK, _zcopy_body, None)
    plsc.subcore_barrier()

    def _edge_loop(table_hbm):
        q = 40
        gs = (gsem, ssem2)

        def gather_start(b):
            for j in range(_CHUNK // q):
                pltpu.async_copy(table_hbm.at[sbuf[b].at[pl.ds(j * q, q)]],
                                 rows[b].at[pl.ds(j * q, q)], gs[j % 2][b])

        def gather_wait(b):
            for j in range(_CHUNK // q):
                pltpu.make_async_copy(table_hbm.at[sbuf[b].at[pl.ds(j * q, q)]],
                                      rows[b].at[pl.ds(j * q, q)],
                                      gs[j % 2][b]).wait()

        src_start(0, 0)
        dst_start(0, 0)
        src_start(1, 1)
        dst_start(1, 1)
        src_wait(0)
        dst_wait(0)
        gather_start(0)

        def _step(t, b, *, idx_next, gather_next):
            gather_wait(b)
            scs = [
                pltpu.async_copy(rows[b].at[pl.ds(j * _Q, _Q)],
                                 acc.at[dbuf[b].at[j]], ssem[b], add=True)
                for j in range(4)
            ]
            if idx_next:
                src_start(t + 2, b)
            if gather_next:
                src_wait(1 - b)
                dst_wait(1 - b)
                gather_start(1 - b)
            for sc in scs:
                sc.wait()
            if idx_next:
                dst_start(t + 2, b)

        def _edge_body(i, _):
            _step(i * 2, 0, idx_next=True, gather_next=True)
            _step(i * 2 + 1, 1, idx_next=True, gather_next=True)
            return _

        lax.fori_loop(0, (_NCHUNK - 3) // 2, _edge_body, None)
        _step(_NCHUNK - 3, 0, idx_next=True, gather_next=True)
        _step(_NCHUNK - 2, 1, idx_next=False, gather_next=True)
        _step(_NCHUNK - 1, 0, idx_next=False, gather_next=False)

    @pl.when(c == 0)
    def _():
        _edge_loop(h_hbm)

    @pl.when(c == 1)
    def _():
        _edge_loop(gc_hbm)

    plsc.subcore_barrier()

    def _wb_body(k, _):
        row = s * _ROWS_PER_SUB + k * _CHUNK
        pltpu.sync_copy(acc.at[pl.ds(row, _CHUNK)], rows0)
        pltpu.sync_copy(rows0, out_hbm.at[c, pl.ds(row, _CHUNK)])
        return _

    lax.fori_loop(0, _ROWS_PER_SUB // _CHUNK, _wb_body, None)


def _post_body(hs_ref, ca_ref, ix_ref, ut_ref, b_ref, h_ref, c_ref):
    sm = jnp.dot(hs_ref[0], ut_ref[...], preferred_element_type=jnp.float32)
    iou = ix_ref[...] + sm + b_ref[...]
    i = jax.nn.sigmoid(iou[:, :DIM])
    o = jax.nn.sigmoid(iou[:, DIM:2 * DIM])
    u = jnp.tanh(iou[:, 2 * DIM:])
    c_new = i * u + ca_ref[0]
    h_ref[...] = o * jnp.tanh(c_new)
    c_ref[...] = c_new


def _post(agg, iou_x, ut, b):
    return pl.pallas_call(
        _post_body,
        grid=(_PGRID,),
        in_specs=[
            pl.BlockSpec((1, _PROWS, DIM), lambda i: (0, i, 0)),
            pl.BlockSpec((1, _PROWS, DIM), lambda i: (1, i, 0)),
            pl.BlockSpec((_PROWS, 3 * DIM), lambda i: (i, 0)),
            pl.BlockSpec((DIM, 3 * DIM), lambda i: (0, 0)),
            pl.BlockSpec((1, 3 * DIM), lambda i: (0, 0)),
        ],
        out_specs=[
            pl.BlockSpec((_PROWS, DIM), lambda i: (i, 0)),
            pl.BlockSpec((_PROWS, DIM), lambda i: (i, 0)),
        ],
        out_shape=[
            jax.ShapeDtypeStruct((N, DIM), jnp.float32),
            jax.ShapeDtypeStruct((N, DIM), jnp.float32),
        ],
    )(agg, agg, iou_x, ut, b)


@jax.jit
def kernel(x, h, c, edge_index, W_iouf, U_iou, b_iou, U_f_W, U_f_b):
    src = edge_index[0].astype(jnp.int32)
    dst = edge_index[1].astype(jnp.int32)

    w3t = W_iouf[:3 * DIM].T
    uft = U_f_W.T
    ufb = U_f_b.reshape(1, DIM)
    ut = U_iou.T

    iou_x, gc = _pre(x, h, c, w3t, uft, ufb)
    agg = _build_sc_segment_sum()(h, gc, src, dst)
    h_new, c_new = _post(agg, iou_x, ut, b_iou)
    return h_new, c_new


# speedup vs baseline: 1.0076x; 1.0046x over previous
import functools

import jax
import jax.numpy as jnp
from jax import lax
from jax.experimental import pallas as pl
from jax.experimental.pallas import tpu as pltpu
from jax.experimental.pallas import tpu_sc as plsc

N = 10000
E = 320000
DIM = 128


_ROWS = 400
_GRID = N // _ROWS
_PROWS = 400
_PGRID = N // _PROWS


def _pre_body(x_ref, h_ref, c_ref, w3t_ref, uft_ref, ufb_ref, iou_ref, gc_ref):
    fh = jnp.dot(h_ref[...], uft_ref[...], preferred_element_type=jnp.float32)
    g = jax.nn.sigmoid(fh + ufb_ref[...])
    gc_ref[...] = g * c_ref[...]
    iou_ref[...] = jnp.dot(x_ref[...], w3t_ref[...],
                           preferred_element_type=jnp.float32)


def _pre(x, h, c, w3t, uft, ufb):
    return pl.pallas_call(
        _pre_body,
        grid=(_GRID,),
        in_specs=[
            pl.BlockSpec((_ROWS, DIM), lambda i: (i, 0)),
            pl.BlockSpec((_ROWS, DIM), lambda i: (i, 0)),
            pl.BlockSpec((_ROWS, DIM), lambda i: (i, 0)),
            pl.BlockSpec((DIM, 3 * DIM), lambda i: (0, 0)),
            pl.BlockSpec((DIM, DIM), lambda i: (0, 0)),
            pl.BlockSpec((1, DIM), lambda i: (0, 0)),
        ],
        out_specs=[
            pl.BlockSpec((_ROWS, 3 * DIM), lambda i: (i, 0)),
            pl.BlockSpec((_ROWS, DIM), lambda i: (i, 0)),
        ],
        out_shape=[
            jax.ShapeDtypeStruct((N, 3 * DIM), jnp.float32),
            jax.ShapeDtypeStruct((N, DIM), jnp.float32),
        ],
    )(x, h, c, w3t, uft, ufb)


_NSUB = 16
_CHUNK = 160
_NCHUNK = 125
_EDGES_PER_SUB = _NCHUNK * _CHUNK
_APAD = 10240
_ROWS_PER_SUB = _APAD // _NSUB

@functools.cache
def _build_sc_segment_sum():
    mesh = plsc.VectorSubcoreMesh(core_axis_name="c", subcore_axis_name="s")
    return pl.kernel(
        _sc_segment_sum_body,
        out_type=jax.ShapeDtypeStruct((2, _APAD, DIM), jnp.float32),
        mesh=mesh,
        scratch_types=[
            pltpu.VMEM((_CHUNK,), jnp.int32),
            pltpu.VMEM((_CHUNK,), jnp.int32),
            pltpu.VMEM((_CHUNK // 2,), jnp.int32),
            pltpu.VMEM((_CHUNK // 2,), jnp.int32),
            pltpu.VMEM((_CHUNK // 2,), jnp.int32),
            pltpu.VMEM((_CHUNK // 2,), jnp.int32),
            pltpu.VMEM((_CHUNK, DIM), jnp.float32),
            pltpu.VMEM((_CHUNK, DIM), jnp.float32),
            pltpu.VMEM_SHARED((_APAD, DIM), jnp.float32),
        ] + [pltpu.SemaphoreType.DMA] * 10,
    )


def _sc_segment_sum_body(h_hbm, gc_hbm, src_hbm, dst_hbm, out_hbm,
                         sbuf0, sbuf1, dbuf0l, dbuf0h, dbuf1l, dbuf1h,
                         rows0, rows1, acc,
                         gsem0, gsem1, ssem0, ssem1, is0, is1, id0, id1,
                         g2sem0, g2sem1):
    c = lax.axis_index("c")
    s = lax.axis_index("s")
    sbuf = (sbuf0, sbuf1)
    dbuf = ((dbuf0l, dbuf0h), (dbuf1l, dbuf1h))
    rows = (rows0, rows1)
    gsem = (gsem0, gsem1)
    ssem = (ssem0, ssem1)
    ssem2 = (g2sem0, g2sem1)
    isem = (is0, is1)
    idsem = (id0, id1)
    base = s * _EDGES_PER_SUB
    _H = _CHUNK // 2

    def src_start(t, b):
        pltpu.async_copy(src_hbm.at[pl.ds(base + t * _CHUNK, _CHUNK)],
                         sbuf[b], isem[b])

    def src_wait(b):
        pltpu.make_async_copy(src_hbm.at[pl.ds(base, _CHUNK)],
                              sbuf[b], isem[b]).wait()

    def dst_start(t, b):
        for j in range(2):
            pltpu.async_copy(
                dst_hbm.at[pl.ds(base + t * _CHUNK + j * _H, _H)],
                dbuf[b][j], idsem[b])

    def dst_wait(b):
        for j in range(2):
            pltpu.make_async_copy(dst_hbm.at[pl.ds(base, _H)],
                                  dbuf[b][j], idsem[b]).wait()

    def _zero_body(i, _):
        r = i // (DIM // 16)
        k = (i % (DIM // 16)) * 16
        rows0[r, pl.ds(k, 16)] = jnp.zeros((16,), jnp.float32)
        return _

    lax.fori_loop(0, _CHUNK * (DIM // 16), _zero_body, None)

    def _zcopy_body(k, _):
        pltpu.sync_copy(rows0,
                        acc.at[pl.ds(s * _ROWS_PER_SUB + k * _CHUNK, _CHUNK)])
        return _

    lax.fori_loop(0, _ROWS_PER_SUB // _CHUNK, _zcopy_body, None)
    plsc.subcore_barrier()

    def _edge_loop(table_hbm):
        q = 40
        gs = (gsem, ssem2)

        def gather_start(b):
            for j in range(_CHUNK // q):
                pltpu.async_copy(table_hbm.at[sbuf[b].at[pl.ds(j * q, q)]],
                                 rows[b].at[pl.ds(j * q, q)], gs[j % 2][b])

        def gather_wait(b):
            for j in range(_CHUNK // q):
                pltpu.make_async_copy(table_hbm.at[sbuf[b].at[pl.ds(j * q, q)]],
                                      rows[b].at[pl.ds(j * q, q)],
                                      gs[j % 2][b]).wait()

        src_start(0, 0)
        dst_start(0, 0)
        src_start(1, 1)
        dst_start(1, 1)
        src_wait(0)
        dst_wait(0)
        gather_start(0)

        def _step(t, b, *, idx_next, gather_next):
            gather_wait(b)
            scs = [
                pltpu.async_copy(rows[b].at[pl.ds(j * _H, _H)],
                                 acc.at[dbuf[b][j]], ssem[b], add=True)
                for j in range(2)
            ]
            if idx_next:
                src_start(t + 2, b)
            if gather_next:
                src_wait(1 - b)
                dst_wait(1 - b)
                gather_start(1 - b)
            for sc in scs:
                sc.wait()
            if idx_next:
                dst_start(t + 2, b)

        def _edge_body(i, _):
            _step(i * 2, 0, idx_next=True, gather_next=True)
            _step(i * 2 + 1, 1, idx_next=True, gather_next=True)
            return _

        lax.fori_loop(0, (_NCHUNK - 3) // 2, _edge_body, None)
        _step(_NCHUNK - 3, 0, idx_next=True, gather_next=True)
        _step(_NCHUNK - 2, 1, idx_next=False, gather_next=True)
        _step(_NCHUNK - 1, 0, idx_next=False, gather_next=False)

    @pl.when(c == 0)
    def _():
        _edge_loop(h_hbm)

    @pl.when(c == 1)
    def _():
        _edge_loop(gc_hbm)

    plsc.subcore_barrier()

    def _wb_body(k, _):
        row = s * _ROWS_PER_SUB + k * _CHUNK
        pltpu.sync_copy(acc.at[pl.ds(row, _CHUNK)], rows0)
        pltpu.sync_copy(rows0, out_hbm.at[c, pl.ds(row, _CHUNK)])
        return _

    lax.fori_loop(0, _ROWS_PER_SUB // _CHUNK, _wb_body, None)


def _post_body(hs_ref, ca_ref, ix_ref, ut_ref, b_ref, h_ref, c_ref):
    sm = jnp.dot(hs_ref[0], ut_ref[...], preferred_element_type=jnp.float32)
    iou = ix_ref[...] + sm + b_ref[...]
    i = jax.nn.sigmoid(iou[:, :DIM])
    o = jax.nn.sigmoid(iou[:, DIM:2 * DIM])
    u = jnp.tanh(iou[:, 2 * DIM:])
    c_new = i * u + ca_ref[0]
    h_ref[...] = o * jnp.tanh(c_new)
    c_ref[...] = c_new


def _post(agg, iou_x, ut, b):
    return pl.pallas_call(
        _post_body,
        grid=(_PGRID,),
        in_specs=[
            pl.BlockSpec((1, _PROWS, DIM), lambda i: (0, i, 0)),
            pl.BlockSpec((1, _PROWS, DIM), lambda i: (1, i, 0)),
            pl.BlockSpec((_PROWS, 3 * DIM), lambda i: (i, 0)),
            pl.BlockSpec((DIM, 3 * DIM), lambda i: (0, 0)),
            pl.BlockSpec((1, 3 * DIM), lambda i: (0, 0)),
        ],
        out_specs=[
            pl.BlockSpec((_PROWS, DIM), lambda i: (i, 0)),
            pl.BlockSpec((_PROWS, DIM), lambda i: (i, 0)),
        ],
        out_shape=[
            jax.ShapeDtypeStruct((N, DIM), jnp.float32),
            jax.ShapeDtypeStruct((N, DIM), jnp.float32),
        ],
    )(agg, agg, iou_x, ut, b)


@jax.jit
def kernel(x, h, c, edge_index, W_iouf, U_iou, b_iou, U_f_W, U_f_b):
    src = edge_index[0].astype(jnp.int32)
    dst = edge_index[1].astype(jnp.int32)

    w3t = W_iouf[:3 * DIM].T
    uft = U_f_W.T
    ufb = U_f_b.reshape(1, DIM)
    ut = U_iou.T

    iou_x, gc = _pre(x, h, c, w3t, uft, ufb)
    agg = _build_sc_segment_sum()(h, gc, src, dst)
    h_new, c_new = _post(agg, iou_x, ut, b_iou)
    return h_new, c_new


# speedup vs baseline: 1.0392x; 1.0313x over previous
import functools

import jax
import jax.numpy as jnp
from jax import lax
from jax.experimental import pallas as pl
from jax.experimental.pallas import tpu as pltpu
from jax.experimental.pallas import tpu_sc as plsc

N = 10000
E = 320000
DIM = 128


_ROWS = 400
_GRID = N // _ROWS
_PROWS = 400
_PGRID = N // _PROWS


def _pre_body(h_ref, c_ref, uft_ref, ufb_ref, gc_ref):
    fh = jnp.dot(h_ref[...], uft_ref[...], preferred_element_type=jnp.float32)
    g = jax.nn.sigmoid(fh + ufb_ref[...])
    gc_ref[...] = g * c_ref[...]


def _pre(h, c, uft, ufb):
    return pl.pallas_call(
        _pre_body,
        grid=(_GRID,),
        in_specs=[
            pl.BlockSpec((_ROWS, DIM), lambda i: (i, 0)),
            pl.BlockSpec((_ROWS, DIM), lambda i: (i, 0)),
            pl.BlockSpec((DIM, DIM), lambda i: (0, 0)),
            pl.BlockSpec((1, DIM), lambda i: (0, 0)),
        ],
        out_specs=pl.BlockSpec((_ROWS, DIM), lambda i: (i, 0)),
        out_shape=jax.ShapeDtypeStruct((N, DIM), jnp.float32),
    )(h, c, uft, ufb)


_NSUB = 16
_CHUNK = 160
_NCHUNK = 125
_EDGES_PER_SUB = _NCHUNK * _CHUNK
_APAD = 10240
_ROWS_PER_SUB = _APAD // _NSUB

@functools.cache
def _build_sc_segment_sum():
    mesh = plsc.VectorSubcoreMesh(core_axis_name="c", subcore_axis_name="s")
    return pl.kernel(
        _sc_segment_sum_body,
        out_type=jax.ShapeDtypeStruct((2, _APAD, DIM), jnp.float32),
        mesh=mesh,
        scratch_types=[
            pltpu.VMEM((_CHUNK,), jnp.int32),
            pltpu.VMEM((_CHUNK,), jnp.int32),
            pltpu.VMEM((_CHUNK // 2,), jnp.int32),
            pltpu.VMEM((_CHUNK // 2,), jnp.int32),
            pltpu.VMEM((_CHUNK // 2,), jnp.int32),
            pltpu.VMEM((_CHUNK // 2,), jnp.int32),
            pltpu.VMEM((_CHUNK, DIM), jnp.float32),
            pltpu.VMEM((_CHUNK, DIM), jnp.float32),
            pltpu.VMEM_SHARED((_APAD, DIM), jnp.float32),
        ] + [pltpu.SemaphoreType.DMA] * 10,
    )


def _sc_segment_sum_body(h_hbm, gc_hbm, src_hbm, dst_hbm, out_hbm,
                         sbuf0, sbuf1, dbuf0l, dbuf0h, dbuf1l, dbuf1h,
                         rows0, rows1, acc,
                         gsem0, gsem1, ssem0, ssem1, is0, is1, id0, id1,
                         g2sem0, g2sem1):
    c = lax.axis_index("c")
    s = lax.axis_index("s")
    sbuf = (sbuf0, sbuf1)
    dbuf = ((dbuf0l, dbuf0h), (dbuf1l, dbuf1h))
    rows = (rows0, rows1)
    gsem = (gsem0, gsem1)
    ssem = (ssem0, ssem1)
    ssem2 = (g2sem0, g2sem1)
    isem = (is0, is1)
    idsem = (id0, id1)
    base = s * _EDGES_PER_SUB
    _H = _CHUNK // 2

    def src_start(t, b):
        pltpu.async_copy(src_hbm.at[pl.ds(base + t * _CHUNK, _CHUNK)],
                         sbuf[b], isem[b])

    def src_wait(b):
        pltpu.make_async_copy(src_hbm.at[pl.ds(base, _CHUNK)],
                              sbuf[b], isem[b]).wait()

    def dst_start(t, b):
        for j in range(2):
            pltpu.async_copy(
                dst_hbm.at[pl.ds(base + t * _CHUNK + j * _H, _H)],
                dbuf[b][j], idsem[b])

    def dst_wait(b):
        for j in range(2):
            pltpu.make_async_copy(dst_hbm.at[pl.ds(base, _H)],
                                  dbuf[b][j], idsem[b]).wait()

    def _zero_body(i, _):
        r = i // (DIM // 16)
        k = (i % (DIM // 16)) * 16
        rows0[r, pl.ds(k, 16)] = jnp.zeros((16,), jnp.float32)
        return _

    lax.fori_loop(0, _CHUNK * (DIM // 16), _zero_body, None)

    def _zcopy_body(k, _):
        pltpu.sync_copy(rows0,
                        acc.at[pl.ds(s * _ROWS_PER_SUB + k * _CHUNK, _CHUNK)])
        return _

    lax.fori_loop(0, _ROWS_PER_SUB // _CHUNK, _zcopy_body, None)
    plsc.subcore_barrier()

    def _edge_loop(table_hbm):
        q = 40
        gs = (gsem, ssem2)

        def gather_start(b):
            for j in range(_CHUNK // q):
                pltpu.async_copy(table_hbm.at[sbuf[b].at[pl.ds(j * q, q)]],
                                 rows[b].at[pl.ds(j * q, q)], gs[j % 2][b])

        def gather_wait(b):
            for j in range(_CHUNK // q):
                pltpu.make_async_copy(table_hbm.at[sbuf[b].at[pl.ds(j * q, q)]],
                                      rows[b].at[pl.ds(j * q, q)],
                                      gs[j % 2][b]).wait()

        src_start(0, 0)
        dst_start(0, 0)
        src_start(1, 1)
        dst_start(1, 1)
        src_wait(0)
        dst_wait(0)
        gather_start(0)

        def _step(t, b, *, idx_next, gather_next):
            gather_wait(b)
            scs = [
                pltpu.async_copy(rows[b].at[pl.ds(j * _H, _H)],
                                 acc.at[dbuf[b][j]], ssem[b], add=True)
                for j in range(2)
            ]
            if idx_next:
                src_start(t + 2, b)
            if gather_next:
                src_wait(1 - b)
                dst_wait(1 - b)
                gather_start(1 - b)
            for sc in scs:
                sc.wait()
            if idx_next:
                dst_start(t + 2, b)

        def _edge_body(i, _):
            _step(i * 2, 0, idx_next=True, gather_next=True)
            _step(i * 2 + 1, 1, idx_next=True, gather_next=True)
            return _

        lax.fori_loop(0, (_NCHUNK - 3) // 2, _edge_body, None)
        _step(_NCHUNK - 3, 0, idx_next=True, gather_next=True)
        _step(_NCHUNK - 2, 1, idx_next=False, gather_next=True)
        _step(_NCHUNK - 1, 0, idx_next=False, gather_next=False)

    @pl.when(c == 0)
    def _():
        _edge_loop(h_hbm)

    @pl.when(c == 1)
    def _():
        _edge_loop(gc_hbm)

    plsc.subcore_barrier()

    def _wb_body(k, _):
        row = s * _ROWS_PER_SUB + k * _CHUNK
        pltpu.sync_copy(acc.at[pl.ds(row, _CHUNK)], rows0)
        pltpu.sync_copy(rows0, out_hbm.at[c, pl.ds(row, _CHUNK)])
        return _

    lax.fori_loop(0, _ROWS_PER_SUB // _CHUNK, _wb_body, None)


def _post_body(hs_ref, ca_ref, x_ref, w3t_ref, ut_ref, b_ref, h_ref, c_ref):
    sm = jnp.dot(hs_ref[0], ut_ref[...], preferred_element_type=jnp.float32)
    ix = jnp.dot(x_ref[...], w3t_ref[...], preferred_element_type=jnp.float32)
    iou = ix + sm + b_ref[...]
    i = jax.nn.sigmoid(iou[:, :DIM])
    o = jax.nn.sigmoid(iou[:, DIM:2 * DIM])
    u = jnp.tanh(iou[:, 2 * DIM:])
    c_new = i * u + ca_ref[0]
    h_ref[...] = o * jnp.tanh(c_new)
    c_ref[...] = c_new


def _post(agg, x, w3t, ut, b):
    return pl.pallas_call(
        _post_body,
        grid=(_PGRID,),
        in_specs=[
            pl.BlockSpec((1, _PROWS, DIM), lambda i: (0, i, 0)),
            pl.BlockSpec((1, _PROWS, DIM), lambda i: (1, i, 0)),
            pl.BlockSpec((_PROWS, DIM), lambda i: (i, 0)),
            pl.BlockSpec((DIM, 3 * DIM), lambda i: (0, 0)),
            pl.BlockSpec((DIM, 3 * DIM), lambda i: (0, 0)),
            pl.BlockSpec((1, 3 * DIM), lambda i: (0, 0)),
        ],
        out_specs=[
            pl.BlockSpec((_PROWS, DIM), lambda i: (i, 0)),
            pl.BlockSpec((_PROWS, DIM), lambda i: (i, 0)),
        ],
        out_shape=[
            jax.ShapeDtypeStruct((N, DIM), jnp.float32),
            jax.ShapeDtypeStruct((N, DIM), jnp.float32),
        ],
    )(agg, agg, x, w3t, ut, b)


@jax.jit
def kernel(x, h, c, edge_index, W_iouf, U_iou, b_iou, U_f_W, U_f_b):
    src = edge_index[0].astype(jnp.int32)
    dst = edge_index[1].astype(jnp.int32)

    w3t = W_iouf[:3 * DIM].T
    uft = U_f_W.T
    ufb = U_f_b.reshape(1, DIM)
    ut = U_iou.T

    gc = _pre(h, c, uft, ufb)
    agg = _build_sc_segment_sum()(h, gc, src, dst)
    h_new, c_new = _post(agg, x, w3t, ut, b_iou)
    return h_new, c_new


# speedup vs baseline: 1.1138x; 1.0718x over previous
import functools

import jax
import jax.numpy as jnp
from jax import lax
from jax.experimental import pallas as pl
from jax.experimental.pallas import tpu as pltpu
from jax.experimental.pallas import tpu_sc as plsc

N = 10000
E = 320000
DIM = 128


_ROWS = 2000
_GRID = N // _ROWS
_PROWS = 2000
_PGRID = N // _PROWS


def _pre_body(h_ref, c_ref, uft_ref, ufb_ref, gc_ref):
    fh = jnp.dot(h_ref[...], uft_ref[...], preferred_element_type=jnp.float32)
    g = jax.nn.sigmoid(fh + ufb_ref[...])
    gc_ref[...] = g * c_ref[...]


def _pre(h, c, uft, ufb):
    return pl.pallas_call(
        _pre_body,
        grid=(_GRID,),
        in_specs=[
            pl.BlockSpec((_ROWS, DIM), lambda i: (i, 0)),
            pl.BlockSpec((_ROWS, DIM), lambda i: (i, 0)),
            pl.BlockSpec((DIM, DIM), lambda i: (0, 0)),
            pl.BlockSpec((1, DIM), lambda i: (0, 0)),
        ],
        out_specs=pl.BlockSpec((_ROWS, DIM), lambda i: (i, 0)),
        out_shape=jax.ShapeDtypeStruct((N, DIM), jnp.float32),
    )(h, c, uft, ufb)


_NSUB = 16
_CHUNK = 160
_NCHUNK = 125
_EDGES_PER_SUB = _NCHUNK * _CHUNK
_APAD = 10240
_ROWS_PER_SUB = _APAD // _NSUB

@functools.cache
def _build_sc_segment_sum():
    mesh = plsc.VectorSubcoreMesh(core_axis_name="c", subcore_axis_name="s")
    return pl.kernel(
        _sc_segment_sum_body,
        out_type=jax.ShapeDtypeStruct((2, _APAD, DIM), jnp.float32),
        mesh=mesh,
        scratch_types=[
            pltpu.VMEM((_CHUNK,), jnp.int32),
            pltpu.VMEM((_CHUNK,), jnp.int32),
            pltpu.VMEM((_CHUNK // 2,), jnp.int32),
            pltpu.VMEM((_CHUNK // 2,), jnp.int32),
            pltpu.VMEM((_CHUNK // 2,), jnp.int32),
            pltpu.VMEM((_CHUNK // 2,), jnp.int32),
            pltpu.VMEM((_CHUNK, DIM), jnp.float32),
            pltpu.VMEM((_CHUNK, DIM), jnp.float32),
            pltpu.VMEM_SHARED((_APAD, DIM), jnp.float32),
        ] + [pltpu.SemaphoreType.DMA] * 10,
    )


def _sc_segment_sum_body(h_hbm, gc_hbm, src_hbm, dst_hbm, out_hbm,
                         sbuf0, sbuf1, dbuf0l, dbuf0h, dbuf1l, dbuf1h,
                         rows0, rows1, acc,
                         gsem0, gsem1, ssem0, ssem1, is0, is1, id0, id1,
                         g2sem0, g2sem1):
    c = lax.axis_index("c")
    s = lax.axis_index("s")
    sbuf = (sbuf0, sbuf1)
    dbuf = ((dbuf0l, dbuf0h), (dbuf1l, dbuf1h))
    rows = (rows0, rows1)
    gsem = (gsem0, gsem1)
    ssem = (ssem0, ssem1)
    ssem2 = (g2sem0, g2sem1)
    isem = (is0, is1)
    idsem = (id0, id1)
    base = s * _EDGES_PER_SUB
    _H = _CHUNK // 2

    def src_start(t, b):
        pltpu.async_copy(src_hbm.at[pl.ds(base + t * _CHUNK, _CHUNK)],
                         sbuf[b], isem[b])

    def src_wait(b):
        pltpu.make_async_copy(src_hbm.at[pl.ds(base, _CHUNK)],
                              sbuf[b], isem[b]).wait()

    def dst_start(t, b):
        for j in range(2):
            pltpu.async_copy(
                dst_hbm.at[pl.ds(base + t * _CHUNK + j * _H, _H)],
                dbuf[b][j], idsem[b])

    def dst_wait(b):
        for j in range(2):
            pltpu.make_async_copy(dst_hbm.at[pl.ds(base, _H)],
                                  dbuf[b][j], idsem[b]).wait()

    src_start(0, 0)
    dst_start(0, 0)
    src_start(1, 1)
    dst_start(1, 1)

    def _zero_body(i, _):
        r = i // (DIM // 16)
        k = (i % (DIM // 16)) * 16
        rows0[r, pl.ds(k, 16)] = jnp.zeros((16,), jnp.float32)
        return _

    lax.fori_loop(0, _CHUNK * (DIM // 16), _zero_body, None)

    def _zcopy_body(k, _):
        pltpu.sync_copy(rows0,
                        acc.at[pl.ds(s * _ROWS_PER_SUB + k * _CHUNK, _CHUNK)])
        return _

    lax.fori_loop(0, _ROWS_PER_SUB // _CHUNK, _zcopy_body, None)
    plsc.subcore_barrier()

    def _edge_loop(table_hbm):
        q = 40
        gs = (gsem, ssem2)

        def gather_start(b):
            for j in range(_CHUNK // q):
                pltpu.async_copy(table_hbm.at[sbuf[b].at[pl.ds(j * q, q)]],
                                 rows[b].at[pl.ds(j * q, q)], gs[j % 2][b])

        def gather_wait(b):
            for j in range(_CHUNK // q):
                pltpu.make_async_copy(table_hbm.at[sbuf[b].at[pl.ds(j * q, q)]],
                                      rows[b].at[pl.ds(j * q, q)],
                                      gs[j % 2][b]).wait()

        src_wait(0)
        dst_wait(0)
        gather_start(0)

        def _step(t, b, *, idx_next, gather_next):
            gather_wait(b)
            scs = [
                pltpu.async_copy(rows[b].at[pl.ds(j * _H, _H)],
                                 acc.at[dbuf[b][j]], ssem[b], add=True)
                for j in range(2)
            ]
            if idx_next:
                src_start(t + 2, b)
            if gather_next:
                src_wait(1 - b)
                dst_wait(1 - b)
                gather_start(1 - b)
            for sc in scs:
                sc.wait()
            if idx_next:
                dst_start(t + 2, b)

        def _edge_body(i, _):
            _step(i * 2, 0, idx_next=True, gather_next=True)
            _step(i * 2 + 1, 1, idx_next=True, gather_next=True)
            return _

        lax.fori_loop(0, (_NCHUNK - 3) // 2, _edge_body, None)
        _step(_NCHUNK - 3, 0, idx_next=True, gather_next=True)
        _step(_NCHUNK - 2, 1, idx_next=False, gather_next=True)
        _step(_NCHUNK - 1, 0, idx_next=False, gather_next=False)

    @pl.when(c == 0)
    def _():
        _edge_loop(h_hbm)

    @pl.when(c == 1)
    def _():
        _edge_loop(gc_hbm)

    plsc.subcore_barrier()

    def _wb_body(k, _):
        row = s * _ROWS_PER_SUB + k * _CHUNK
        pltpu.sync_copy(acc.at[pl.ds(row, _CHUNK)], rows0)
        pltpu.sync_copy(rows0, out_hbm.at[c, pl.ds(row, _CHUNK)])
        return _

    lax.fori_loop(0, _ROWS_PER_SUB // _CHUNK, _wb_body, None)


def _post_body(hs_ref, ca_ref, x_ref, w3t_ref, ut_ref, b_ref, h_ref, c_ref):
    sm = jnp.dot(hs_ref[0], ut_ref[...], preferred_element_type=jnp.float32)
    ix = jnp.dot(x_ref[...], w3t_ref[...], preferred_element_type=jnp.float32)
    iou = ix + sm + b_ref[...]
    i = jax.nn.sigmoid(iou[:, :DIM])
    o = jax.nn.sigmoid(iou[:, DIM:2 * DIM])
    u = jnp.tanh(iou[:, 2 * DIM:])
    c_new = i * u + ca_ref[0]
    h_ref[...] = o * jnp.tanh(c_new)
    c_ref[...] = c_new


def _post(agg, x, w3t, ut, b):
    return pl.pallas_call(
        _post_body,
        grid=(_PGRID,),
        in_specs=[
            pl.BlockSpec((1, _PROWS, DIM), lambda i: (0, i, 0)),
            pl.BlockSpec((1, _PROWS, DIM), lambda i: (1, i, 0)),
            pl.BlockSpec((_PROWS, DIM), lambda i: (i, 0)),
            pl.BlockSpec((DIM, 3 * DIM), lambda i: (0, 0)),
            pl.BlockSpec((DIM, 3 * DIM), lambda i: (0, 0)),
            pl.BlockSpec((1, 3 * DIM), lambda i: (0, 0)),
        ],
        out_specs=[
            pl.BlockSpec((_PROWS, DIM), lambda i: (i, 0)),
            pl.BlockSpec((_PROWS, DIM), lambda i: (i, 0)),
        ],
        out_shape=[
            jax.ShapeDtypeStruct((N, DIM), jnp.float32),
            jax.ShapeDtypeStruct((N, DIM), jnp.float32),
        ],
    )(agg, agg, x, w3t, ut, b)


@jax.jit
def kernel(x, h, c, edge_index, W_iouf, U_iou, b_iou, U_f_W, U_f_b):
    src = edge_index[0].astype(jnp.int32)
    dst = edge_index[1].astype(jnp.int32)

    w3t = W_iouf[:3 * DIM].T
    uft = U_f_W.T
    ufb = U_f_b.reshape(1, DIM)
    ut = U_iou.T

    gc = _pre(h, c, uft, ufb)
    agg = _build_sc_segment_sum()(h, gc, src, dst)
    h_new, c_new = _post(agg, x, w3t, ut, b_iou)
    return h_new, c_new


# speedup vs baseline: 1.1321x; 1.0164x over previous
import functools

import jax
import jax.numpy as jnp
from jax import lax
from jax.experimental import pallas as pl
from jax.experimental.pallas import tpu as pltpu
from jax.experimental.pallas import tpu_sc as plsc

N = 10000
E = 320000
DIM = 128


_ROWS = 2000
_GRID = N // _ROWS
_PROWS = 2000
_PGRID = N // _PROWS


def _pre_body(h_ref, c_ref, uft_ref, ufb_ref, gc_ref):
    fh = jnp.dot(h_ref[...], uft_ref[...], preferred_element_type=jnp.float32)
    g = jax.nn.sigmoid(fh + ufb_ref[...])
    gc_ref[...] = g * c_ref[...]


def _pre(h, c, uft, ufb):
    return pl.pallas_call(
        _pre_body,
        grid=(_GRID,),
        in_specs=[
            pl.BlockSpec((_ROWS, DIM), lambda i: (i, 0)),
            pl.BlockSpec((_ROWS, DIM), lambda i: (i, 0)),
            pl.BlockSpec((DIM, DIM), lambda i: (0, 0)),
            pl.BlockSpec((1, DIM), lambda i: (0, 0)),
        ],
        out_specs=pl.BlockSpec((_ROWS, DIM), lambda i: (i, 0)),
        out_shape=jax.ShapeDtypeStruct((N, DIM), jnp.float32),
    )(h, c, uft, ufb)


_NSUB = 16
_CHUNK = 160
_NCHUNK = 125
_EDGES_PER_SUB = _NCHUNK * _CHUNK
_APAD = 10240
_ROWS_PER_SUB = _APAD // _NSUB

@functools.cache
def _build_sc_segment_sum():
    mesh = plsc.VectorSubcoreMesh(core_axis_name="c", subcore_axis_name="s")
    return pl.kernel(
        _sc_segment_sum_body,
        out_type=jax.ShapeDtypeStruct((2, _APAD, DIM), jnp.float32),
        mesh=mesh,
        scratch_types=[
            pltpu.VMEM((_CHUNK,), jnp.int32),
            pltpu.VMEM((_CHUNK,), jnp.int32),
            pltpu.VMEM((_CHUNK // 2,), jnp.int32),
            pltpu.VMEM((_CHUNK // 2,), jnp.int32),
            pltpu.VMEM((_CHUNK // 2,), jnp.int32),
            pltpu.VMEM((_CHUNK // 2,), jnp.int32),
            pltpu.VMEM((_CHUNK, DIM), jnp.float32),
            pltpu.VMEM((_CHUNK, DIM), jnp.float32),
            pltpu.VMEM_SHARED((_APAD, DIM), jnp.float32),
        ] + [pltpu.SemaphoreType.DMA] * 10,
    )


def _sc_segment_sum_body(h_hbm, gc_hbm, src_hbm, dst_hbm, out_hbm,
                         sbuf0, sbuf1, dbuf0l, dbuf0h, dbuf1l, dbuf1h,
                         rows0, rows1, acc,
                         gsem0, gsem1, ssem0, ssem1, is0, is1, id0, id1,
                         g2sem0, g2sem1):
    c = lax.axis_index("c")
    s = lax.axis_index("s")
    sbuf = (sbuf0, sbuf1)
    dbuf = ((dbuf0l, dbuf0h), (dbuf1l, dbuf1h))
    rows = (rows0, rows1)
    gsem = (gsem0, gsem1)
    ssem = (ssem0, ssem1)
    ssem2 = (g2sem0, g2sem1)
    isem = (is0, is1)
    idsem = (id0, id1)
    base = s * _EDGES_PER_SUB
    _H = _CHUNK // 2

    def src_start(t, b):
        pltpu.async_copy(src_hbm.at[pl.ds(base + t * _CHUNK, _CHUNK)],
                         sbuf[b], isem[b])

    def src_wait(b):
        pltpu.make_async_copy(src_hbm.at[pl.ds(base, _CHUNK)],
                              sbuf[b], isem[b]).wait()

    def dst_start(t, b):
        for j in range(2):
            pltpu.async_copy(
                dst_hbm.at[pl.ds(base + t * _CHUNK + j * _H, _H)],
                dbuf[b][j], idsem[b])

    def dst_wait(b):
        for j in range(2):
            pltpu.make_async_copy(dst_hbm.at[pl.ds(base, _H)],
                                  dbuf[b][j], idsem[b]).wait()

    src_start(0, 0)
    dst_start(0, 0)
    src_start(1, 1)
    dst_start(1, 1)

    def _zero_body(i, _):
        r = i // (DIM // 16)
        k = (i % (DIM // 16)) * 16
        rows0[r, pl.ds(k, 16)] = jnp.zeros((16,), jnp.float32)
        return _

    lax.fori_loop(0, _CHUNK * (DIM // 16), _zero_body, None)

    def _zcopy_body(k, _):
        pltpu.sync_copy(rows0,
                        acc.at[pl.ds(s * _ROWS_PER_SUB + k * _CHUNK, _CHUNK)])
        return _

    lax.fori_loop(0, _ROWS_PER_SUB // _CHUNK, _zcopy_body, None)
    plsc.subcore_barrier()

    def _edge_loop(table_hbm):
        q = 40
        gs = (gsem, ssem2)

        def gather_start(b):
            for j in range(_CHUNK // q):
                pltpu.async_copy(table_hbm.at[sbuf[b].at[pl.ds(j * q, q)]],
                                 rows[b].at[pl.ds(j * q, q)], gs[j % 2][b])

        def gather_wait(b, js):
            for j in js:
                pltpu.make_async_copy(table_hbm.at[sbuf[b].at[pl.ds(j * q, q)]],
                                      rows[b].at[pl.ds(j * q, q)],
                                      gs[j % 2][b]).wait()

        src_wait(0)
        dst_wait(0)
        gather_start(0)

        def _step(t, b, *, idx_next, gather_next):
            scs = []
            for j in range(2):
                gather_wait(b, (2 * j, 2 * j + 1))
                scs.append(
                    pltpu.async_copy(rows[b].at[pl.ds(j * _H, _H)],
                                     acc.at[dbuf[b][j]], ssem[b], add=True))
            if idx_next:
                src_start(t + 2, b)
            if gather_next:
                src_wait(1 - b)
                dst_wait(1 - b)
                gather_start(1 - b)
            for sc in scs:
                sc.wait()
            if idx_next:
                dst_start(t + 2, b)

        def _edge_body(i, _):
            _step(i * 2, 0, idx_next=True, gather_next=True)
            _step(i * 2 + 1, 1, idx_next=True, gather_next=True)
            return _

        lax.fori_loop(0, (_NCHUNK - 3) // 2, _edge_body, None)
        _step(_NCHUNK - 3, 0, idx_next=True, gather_next=True)
        _step(_NCHUNK - 2, 1, idx_next=False, gather_next=True)
        _step(_NCHUNK - 1, 0, idx_next=False, gather_next=False)

    @pl.when(c == 0)
    def _():
        _edge_loop(h_hbm)

    @pl.when(c == 1)
    def _():
        _edge_loop(gc_hbm)

    plsc.subcore_barrier()

    def _wb_body(k, _):
        row = s * _ROWS_PER_SUB + k * _CHUNK
        pltpu.sync_copy(acc.at[pl.ds(row, _CHUNK)], rows0)
        pltpu.sync_copy(rows0, out_hbm.at[c, pl.ds(row, _CHUNK)])
        return _

    lax.fori_loop(0, _ROWS_PER_SUB // _CHUNK, _wb_body, None)


def _post_body(hs_ref, ca_ref, x_ref, w3t_ref, ut_ref, b_ref, h_ref, c_ref):
    sm = jnp.dot(hs_ref[0], ut_ref[...], preferred_element_type=jnp.float32)
    ix = jnp.dot(x_ref[...], w3t_ref[...], preferred_element_type=jnp.float32)
    iou = ix + sm + b_ref[...]
    i = jax.nn.sigmoid(iou[:, :DIM])
    o = jax.nn.sigmoid(iou[:, DIM:2 * DIM])
    u = jnp.tanh(iou[:, 2 * DIM:])
    c_new = i * u + ca_ref[0]
    h_ref[...] = o * jnp.tanh(c_new)
    c_ref[...] = c_new


def _post(agg, x, w3t, ut, b):
    return pl.pallas_call(
        _post_body,
        grid=(_PGRID,),
        in_specs=[
            pl.BlockSpec((1, _PROWS, DIM), lambda i: (0, i, 0)),
            pl.BlockSpec((1, _PROWS, DIM), lambda i: (1, i, 0)),
            pl.BlockSpec((_PROWS, DIM), lambda i: (i, 0)),
            pl.BlockSpec((DIM, 3 * DIM), lambda i: (0, 0)),
            pl.BlockSpec((DIM, 3 * DIM), lambda i: (0, 0)),
            pl.BlockSpec((1, 3 * DIM), lambda i: (0, 0)),
        ],
        out_specs=[
            pl.BlockSpec((_PROWS, DIM), lambda i: (i, 0)),
            pl.BlockSpec((_PROWS, DIM), lambda i: (i, 0)),
        ],
        out_shape=[
            jax.ShapeDtypeStruct((N, DIM), jnp.float32),
            jax.ShapeDtypeStruct((N, DIM), jnp.float32),
        ],
    )(agg, agg, x, w3t, ut, b)


@jax.jit
def kernel(x, h, c, edge_index, W_iouf, U_iou, b_iou, U_f_W, U_f_b):
    src = edge_index[0].astype(jnp.int32)
    dst = edge_index[1].astype(jnp.int32)

    w3t = W_iouf[:3 * DIM].T
    uft = U_f_W.T
    ufb = U_f_b.reshape(1, DIM)
    ut = U_iou.T

    gc = _pre(h, c, uft, ufb)
    agg = _build_sc_segment_sum()(h, gc, src, dst)
    h_new, c_new = _post(agg, x, w3t, ut, b_iou)
    return h_new, c_new
